# baseline (speedup 1.0000x reference)
import functools
from functools import partial

import jax
import jax.numpy as jnp
from jax import lax
from jax.experimental import pallas as pl
from jax.experimental.pallas import tpu as pltpu
from jax.experimental.pallas import tpu_sc as plsc

N = 10000
E = 320000
DIM = 128
AGG = 64
CLS = 32
ROUNDS = 20

NC = 2
NS = 16
NW = NC * NS
CH = 128
CPW = 2 * (-(-E // (NW * CH * 2)))
NGRP = CPW // 2
CPAD = CPW + 2
EPAD = NW * CPAD * CH
NACC = 10112
RPS = NACC // NS

RBLK = 2000


def _msg_body(pre_hbm, src_hbm, cmul_hbm, slot_hbm, fi_hbm, zeros_hbm,
              bv_hbm, out_hbm,
              acc, src_v, c_v, sl_v, fi_v, rows_v, fb_v, bv_v, srows_v, sems):
    cid = lax.axis_index("c")
    sid = lax.axis_index("s")
    wid = cid * NS + sid

    pltpu.sync_copy(zeros_hbm.at[pl.ds(sid * RPS, RPS)],
                    acc.at[pl.ds(sid * RPS, RPS)])
    plsc.subcore_barrier()

    zero16 = jnp.zeros((16,), jnp.float32)

    def chunk(j, P):
        pltpu.sync_copy(src_hbm.at[wid, j], src_v)
        pltpu.async_copy(pre_hbm.at[src_v], rows_v, sems[0]).wait()
        pltpu.sync_copy(cmul_hbm.at[wid, j], c_v)
        pltpu.sync_copy(slot_hbm.at[wid, j], sl_v)
        pltpu.sync_copy(fi_hbm.at[wid, j], fi_v)
        for g in range(8):
            cvec = c_v[pl.ds(g * 16, 16)]
            svec = sl_v[pl.ds(g * 16, 16)]
            for t in range(16):
                i = g * 16 + t
                ci = cvec[t]
                si = svec[t]
                newP = []
                for q in range(8):
                    pk = P[q] * ci + rows_v[i, pl.ds(q * 16, 16)]
                    fb_v[si, pl.ds(q * 16, 16)] = pk
                    newP.append(pk)
                P = tuple(newP)
        pltpu.sync_copy(fb_v, acc.at[fi_v])
        return P

    lax.fori_loop(0, CPAD, chunk, (zero16,) * 8)
    plsc.subcore_barrier()

    @pl.when(sid == 0)
    def _combine():
        pltpu.sync_copy(bv_hbm.at[cid], bv_v)
        pltpu.sync_copy(acc.at[pl.ds(N + cid * NS, NS)], srows_v)
        pltpu.sync_copy(srows_v, acc.at[bv_v], add=True)

    plsc.subcore_barrier()

    pltpu.sync_copy(acc.at[pl.ds(sid * RPS, RPS)],
                    out_hbm.at[cid, pl.ds(sid * RPS, RPS)])


_msg_kernel = pl.kernel(
    _msg_body,
    out_type=jax.ShapeDtypeStruct((NC, NACC, DIM), jnp.float32),
    mesh=plsc.VectorSubcoreMesh(core_axis_name="c", subcore_axis_name="s"),
    scratch_types=[
        pltpu.VMEM_SHARED((NACC, DIM), jnp.float32),
        pltpu.VMEM((CH,), jnp.int32),
        pltpu.VMEM((CH,), jnp.float32),
        pltpu.VMEM((CH,), jnp.int32),
        pltpu.VMEM((CH,), jnp.int32),
        pltpu.VMEM((CH, DIM), jnp.float32),
        pltpu.VMEM((CH, DIM), jnp.float32),
        pltpu.VMEM((NS,), jnp.int32),
        pltpu.VMEM((NS, DIM), jnp.float32),
        [pltpu.SemaphoreType.DMA for _ in range(2)],
    ],
)


def _init_body(feats, WiT, bi, W1T, b1, W2T, b2, h_out, pre_out):
    h = jnp.dot(feats[...], WiT[...], preferred_element_type=jnp.float32) + bi[...]
    h_out[...] = h
    a = jax.nn.relu(jnp.dot(h, W1T[...], preferred_element_type=jnp.float32) + b1[...])
    pre_out[...] = jnp.dot(a, W2T[...], preferred_element_type=jnp.float32) + b2[...]


def _fused_body(parts, h_ref, WgiT, WghT, bgi, bgh, W1T, b1, W2T, b2,
                h_out, pre_out):
    x = parts[0] + parts[1]
    h = h_ref[...]
    gi = jnp.dot(x, WgiT[...], preferred_element_type=jnp.float32) + bgi[...]
    gh = jnp.dot(h, WghT[...], preferred_element_type=jnp.float32) + bgh[...]
    r = jax.nn.sigmoid(gi[:, :DIM] + gh[:, :DIM])
    z = jax.nn.sigmoid(gi[:, DIM:2 * DIM] + gh[:, DIM:2 * DIM])
    n = jnp.tanh(gi[:, 2 * DIM:] + r * gh[:, 2 * DIM:])
    hn = (1.0 - z) * n + z * h
    h_out[...] = hn
    a = jax.nn.relu(jnp.dot(hn, W1T[...], preferred_element_type=jnp.float32) + b1[...])
    pre_out[...] = jnp.dot(a, W2T[...], preferred_element_type=jnp.float32) + b2[...]


def _cls_body(h_ref, W1T, b1, W2T, b2, out_ref):
    a = jax.nn.relu(jnp.dot(h_ref[...], W1T[...], preferred_element_type=jnp.float32) + b1[...])
    out_ref[...] = jnp.dot(a, W2T[...], preferred_element_type=jnp.float32) + b2[...]


def _row_spec(d):
    return pl.BlockSpec((RBLK, d), lambda i: (i, 0))


def _full_spec(shape):
    nd = len(shape)
    return pl.BlockSpec(shape, lambda i: (0,) * nd)


def _w(shape):
    return _full_spec(shape)


_GRID = (N // RBLK,)


def _init_call(feats, WiT, bi, W1T, b1, W2T, b2):
    return pl.pallas_call(
        _init_body,
        grid=_GRID,
        in_specs=[_row_spec(4), _w((4, DIM)), _w((1, DIM)),
                  _w((DIM, AGG)), _w((1, AGG)), _w((AGG, DIM)), _w((1, DIM))],
        out_specs=[_row_spec(DIM), _row_spec(DIM)],
        out_shape=[jax.ShapeDtypeStruct((N, DIM), jnp.float32),
                   jax.ShapeDtypeStruct((N, DIM), jnp.float32)],
    )(feats, WiT, bi, W1T, b1, W2T, b2)


def _fused_call(parts, h, WgiT, WghT, bgi, bgh, W1T, b1, W2T, b2):
    return pl.pallas_call(
        _fused_body,
        grid=_GRID,
        in_specs=[pl.BlockSpec((NC, RBLK, DIM), lambda i: (0, i, 0)),

                  _row_spec(DIM),
                  _w((DIM, 3 * DIM)), _w((DIM, 3 * DIM)),
                  _w((1, 3 * DIM)), _w((1, 3 * DIM)),
                  _w((DIM, AGG)), _w((1, AGG)), _w((AGG, DIM)), _w((1, DIM))],
        out_specs=[_row_spec(DIM), _row_spec(DIM)],
        out_shape=[jax.ShapeDtypeStruct((N, DIM), jnp.float32),
                   jax.ShapeDtypeStruct((N, DIM), jnp.float32)],
    )(parts, h, WgiT, WghT, bgi, bgh, W1T, b1, W2T, b2)


def _cls_call(h, W1T, b1, W2T, b2):
    return pl.pallas_call(
        _cls_body,
        grid=_GRID,
        in_specs=[_row_spec(DIM), _w((DIM, CLS)), _w((1, CLS)),
                  _w((CLS, 1)), _w((1, 1))],
        out_specs=[_row_spec(1)],
        out_shape=[jax.ShapeDtypeStruct((N, 1), jnp.float32)],
    )(h, W1T, b1, W2T, b2)[0]


EPW = E // NW
PPW = CPAD * CH - EPW


def _pad_idx(idx, dummy_vals):
    pad = jnp.broadcast_to(dummy_vals[None, :], (NW, PPW))
    return jnp.concatenate([idx.reshape(NW, EPW), pad], axis=1
                           ).reshape(NW, CPAD, CH)


_WS = CPAD * CH
_S = NW * _WS
_THROW = NACC - 1


def _sorted_dir(dst, src):
    dst_pad = (N + NW + (jnp.arange(PPW) % (NACC - N - NW))).astype(jnp.int32)
    src_pad = (jnp.arange(PPW) % N).astype(jnp.int32)
    w_of = (jnp.arange(E) // EPW).astype(jnp.int32)
    perm = jnp.argsort(dst, stable=True)
    sdst = dst[perm]
    ssrc = src[perm]
    bvals = sdst[jnp.arange(1, NW) * EPW - 1]
    bv_full = jnp.concatenate([jnp.full((1,), -1, jnp.int32), bvals])[w_of]
    cont = sdst == bv_full
    dst2 = jnp.where(cont, N + w_of, sdst).astype(jnp.int32)
    bv = jnp.concatenate([jnp.full((1,), N, jnp.int32), bvals]).reshape(NC, NS)

    d2p = jnp.concatenate(
        [dst2.reshape(NW, EPW),
         jnp.broadcast_to(dst_pad[None, :], (NW, PPW))], axis=1).reshape(_S)
    srcp = jnp.concatenate(
        [ssrc.reshape(NW, EPW),
         jnp.broadcast_to(src_pad[None, :], (NW, PPW))], axis=1)

    kk = jnp.arange(_S)
    pos = kk % _WS
    prev = jnp.concatenate([jnp.full((1,), -2, jnp.int32), d2p[:-1]])
    new_seg = (pos == 0) | (d2p != prev)
    gcs = jnp.cumsum(new_seg.astype(jnp.int32))
    rank = gcs - gcs[(kk // _WS) * _WS]
    slot = (rank % CH).astype(jnp.int32)
    ends = jnp.concatenate([new_seg[1:], jnp.ones((1,), bool)]) | (pos == _WS - 1)
    chunk_of = pos // CH
    flat_fi = (kk // _WS) * _WS + chunk_of * CH + slot
    fi = jnp.full((_S + 1,), _THROW, jnp.int32)
    fi = fi.at[jnp.where(ends, flat_fi, _S)].set(d2p)[:_S]
    cmul = 1.0 - new_seg.astype(jnp.float32)
    shape3 = (NW, CPAD, CH)
    return (srcp.reshape(shape3), cmul.reshape(shape3),
            slot.reshape(shape3), fi.reshape(shape3), bv)


def kernel(features, edge_index, W_init, b_init, Wf1, bf1, Wf2, bf2,
           Wb1, bb1, Wb2, bb2, Wfg_ih, Wfg_hh, bfg_ih, bfg_hh,
           Wbg_ih, Wbg_hh, bbg_ih, bbg_hh, Wc1, bc1, Wc2, bc2):
    row = edge_index[0]
    col = edge_index[1]
    f_ctl = _sorted_dir(row, col)
    b_ctl = _sorted_dir(col, row)
    zeros_tbl = jnp.zeros((NACC, DIM), jnp.float32)

    r2 = lambda b: b.reshape(1, -1)
    Wf1T, Wf2T = Wf1.T, Wf2.T
    Wb1T, Wb2T = Wb1.T, Wb2.T
    fg = (Wfg_ih.T, Wfg_hh.T, r2(bfg_ih), r2(bfg_hh))
    bg = (Wbg_ih.T, Wbg_hh.T, r2(bbg_ih), r2(bbg_hh))

    h, f_pre = _init_call(features, W_init.T, r2(b_init),
                          Wf1T, r2(bf1), Wf2T, r2(bf2))

    def round_body(_, carry):
        h, f_pre = carry
        f_parts = _msg_kernel(f_pre, f_ctl[0], f_ctl[1], f_ctl[2], f_ctl[3],
                              zeros_tbl, f_ctl[4])
        h, b_pre = _fused_call(f_parts, h, *fg, Wb1T, r2(bb1), Wb2T, r2(bb2))
        b_parts = _msg_kernel(b_pre, b_ctl[0], b_ctl[1], b_ctl[2], b_ctl[3],
                              zeros_tbl, b_ctl[4])
        h, f_pre = _fused_call(b_parts, h, *bg, Wf1T, r2(bf1), Wf2T, r2(bf2))
        return h, f_pre

    h, _ = lax.fori_loop(0, ROUNDS, round_body, (h, f_pre))
    return _cls_call(h, Wc1.T, r2(bc1), Wc2.T, r2(bc2))
